# baseline (speedup 1.0000x reference)
import functools

import jax
import jax.numpy as jnp
from jax import lax
from jax.experimental import pallas as pl
from jax.experimental.pallas import tpu as pltpu
from jax.experimental.pallas import tpu_sc as plsc

_N, _E, _D, _G = 10000, 320000, 128, 64
_NC, _NS = 2, 16
_NW = _NC * _NS
_K = 80
_NCHUNK = 128
_EPW = _NCHUNK * _K
_EPAD = _NW * _EPW
_P = 64
_NTRASH = 16
_NPAD = _N + _NTRASH
_STRIPE = 640
_R = 1000
_NBLK = _N // _R

_F32 = jnp.float32
_HI = lax.Precision.HIGHEST


def _sc_agg(x, src2d, dst2d):
    mesh = plsc.VectorSubcoreMesh(core_axis_name="c", subcore_axis_name="s")

    @functools.partial(
        pl.kernel,
        out_type=jax.ShapeDtypeStruct((_NC, _NPAD, _D), _F32),
        mesh=mesh,
        scratch_types=[
            pltpu.VMEM((_P, _K), jnp.int32),
            pltpu.VMEM((_P, _K), jnp.int32),
            pltpu.VMEM((_K, _D), _F32),
            pltpu.VMEM((_K, _D), _F32),
            pltpu.VMEM_SHARED((_NPAD, _D), _F32),
            pltpu.SemaphoreType.DMA,
            pltpu.SemaphoreType.DMA,
            pltpu.SemaphoreType.DMA,
            pltpu.SemaphoreType.DMA,
        ],
    )
    def agg_kernel(x_hbm, src_hbm, dst_hbm, out_hbm,
                   src_v, dst_v, rows_a, rows_b, acc_sh, g_a, g_b, s_a, s_b):
        cid = lax.axis_index("c")
        sid = lax.axis_index("s")
        wid = sid * _NC + cid
        rows = (rows_a, rows_b)
        gsem = (g_a, g_b)
        ssem = (s_a, s_b)

        @pl.loop(0, _K * (_D // 16))
        def _(i):
            r = i // (_D // 16)
            c = (i % (_D // 16)) * 16
            rows_a[r, pl.ds(c, 16)] = jnp.zeros((16,), _F32)

        base = sid * _STRIPE
        nchk = jnp.where(sid == _NS - 1,
                         (_NPAD - (_NS - 1) * _STRIPE) // 16, _STRIPE // 16)

        @pl.loop(0, nchk)
        def _(t):
            pltpu.sync_copy(rows_a.at[pl.ds(0, 16)],
                            acc_sh.at[pl.ds(base + t * 16, 16)])

        plsc.subcore_barrier()

        for p in range(_NCHUNK // _P):
            pltpu.sync_copy(src_hbm.at[wid, pl.ds(p * _P, _P)], src_v)
            pltpu.sync_copy(dst_hbm.at[wid, pl.ds(p * _P, _P)], dst_v)
            pltpu.async_copy(x_hbm.at[src_v.at[0]], rows_a, g_a)
            pltpu.async_copy(x_hbm.at[src_v.at[1]], rows_b, g_b)

            @pl.loop(0, _P // 2)
            def _(t):
                j = t * 2
                for b in range(2):
                    pltpu.make_async_copy(x_hbm.at[src_v.at[j + b]],
                                          rows[b], gsem[b]).wait()
                    pltpu.async_copy(rows[b], acc_sh.at[dst_v.at[j + b]],
                                     ssem[b], add=True)
                for b in range(2):
                    jb = j + b
                    pltpu.make_async_copy(rows[b], acc_sh.at[dst_v.at[jb]],
                                          ssem[b]).wait()

                    @pl.when(jb + 2 < _P)
                    def _(b=b, jb=jb):
                        pltpu.async_copy(x_hbm.at[src_v.at[jb + 2]],
                                         rows[b], gsem[b])

        plsc.subcore_barrier()

        @pl.loop(0, nchk)
        def _(t):
            pltpu.sync_copy(acc_sh.at[pl.ds(base + t * 16, 16)],
                            out_hbm.at[cid, pl.ds(base + t * 16, 16)])

    return agg_kernel(x, src2d, dst2d)


def _mlp_body(x_ref, a0_ref, a1_ref, w1_ref, b1_ref, w2_ref, b2_ref,
              g_ref, be_ref, o_ref):
    h = x_ref[...] + a0_ref[...] + a1_ref[...]
    h = jnp.maximum(lax.dot(h, w1_ref[...], precision=_HI) + b1_ref[...], 0.0)
    h = lax.dot(h, w2_ref[...], precision=_HI) + b2_ref[...]
    h = jnp.maximum(h, 0.0)
    mu = jnp.mean(h, axis=-1, keepdims=True)
    var = jnp.mean((h - mu) ** 2, axis=-1, keepdims=True)
    o_ref[...] = (h - mu) / jnp.sqrt(var + 1e-5) * g_ref[...] + be_ref[...]


def _tc_mlp(x, a0, a1, w1, b1, w2, b2, g, be):
    row = pl.BlockSpec((_R, _D), lambda i: (i, 0))
    full = pl.BlockSpec((_D, _D), lambda i: (0, 0))
    vec = pl.BlockSpec((1, _D), lambda i: (0, 0))
    return pl.pallas_call(
        _mlp_body,
        grid=(_NBLK,),
        in_specs=[row, row, row, full, vec, full, vec, vec, vec],
        out_specs=row,
        out_shape=jax.ShapeDtypeStruct((_N, _D), _F32),
    )(x, a0, a1, w1, b1.reshape(1, _D), w2, b2.reshape(1, _D),
      g.reshape(1, _D), be.reshape(1, _D))


def _final_body(x_ref, a0_ref, a1_ref, w1_ref, b1_ref, w2_ref, b2_ref,
                batch_ref, wp1_ref, bp1_ref, wp2_ref, bp2_ref,
                o_ref, sums_s, cnt_s):
    i = pl.program_id(0)
    h = x_ref[...] + a0_ref[...] + a1_ref[...]
    h = jnp.maximum(lax.dot(h, w1_ref[...], precision=_HI) + b1_ref[...], 0.0)
    h = lax.dot(h, w2_ref[...], precision=_HI) + b2_ref[...]
    h = jnp.maximum(h, 0.0)

    bb = batch_ref[0]
    oh = (lax.broadcasted_iota(jnp.int32, (_G, _R), 0)
          == jnp.broadcast_to(bb, (_G, _R))).astype(_F32)
    ps = lax.dot(oh, h, precision=_HI)
    pc = jnp.broadcast_to(jnp.sum(oh, axis=1, keepdims=True), (_G, _D))

    @pl.when(i == 0)
    def _():
        sums_s[...] = ps
        cnt_s[...] = pc

    @pl.when(i > 0)
    def _():
        sums_s[...] += ps
        cnt_s[...] += pc

    @pl.when(i == _NBLK - 1)
    def _():
        pooled = sums_s[...] / jnp.maximum(cnt_s[...], 1.0)
        ph = jnp.maximum(lax.dot(pooled, wp1_ref[...], precision=_HI)
                         + bp1_ref[...], 0.0)
        o_ref[...] = lax.dot(ph, wp2_ref[...], precision=_HI) + bp2_ref[...]


def _tc_final(x, a0, a1, w1, b1, w2, b2, batch3d, wp1p, bp1p, wp2p, bp2p):
    row = pl.BlockSpec((_R, _D), lambda i: (i, 0))
    full = pl.BlockSpec((_D, _D), lambda i: (0, 0))
    vec = pl.BlockSpec((1, _D), lambda i: (0, 0))
    bspec = pl.BlockSpec((1, 1, _R), lambda i: (i, 0, 0))
    out = pl.pallas_call(
        _final_body,
        grid=(_NBLK,),
        in_specs=[row, row, row, full, vec, full, vec, bspec,
                  full, vec, full, vec],
        out_specs=pl.BlockSpec((_G, _D), lambda i: (0, 0)),
        out_shape=jax.ShapeDtypeStruct((_G, _D), _F32),
        scratch_shapes=[pltpu.VMEM((_G, _D), _F32), pltpu.VMEM((_G, _D), _F32)],
    )(x, a0, a1, w1, b1.reshape(1, _D), w2, b2.reshape(1, _D), batch3d,
      wp1p, bp1p, wp2p, bp2p)
    return out[:, :1]


def kernel(x, edge_index, batch,
           W1_0, b1_0, W2_0, b2_0,
           W1_1, b1_1, W2_1, b2_1,
           W1_2, b1_2, W2_2, b2_2,
           ln_g_0, ln_b_0, ln_g_1, ln_b_1,
           Wp1, bp1, Wp2, bp2):
    pad = _EPAD - _E
    src2d = jnp.concatenate(
        [edge_index[0], jnp.zeros((pad,), jnp.int32)]
    ).reshape(_NW, _NCHUNK, _K)
    dst2d = jnp.concatenate(
        [edge_index[1], _N + (jnp.arange(pad, dtype=jnp.int32) % _NTRASH)]
    ).reshape(_NW, _NCHUNK, _K)
    batch3d = batch.reshape(_NBLK, 1, _R)

    wp1p = jnp.pad(Wp1, ((0, 0), (0, _D - _D // 2)))
    bp1p = jnp.pad(bp1, (0, _D - _D // 2)).reshape(1, _D)
    wp2p = jnp.pad(Wp2, ((0, _D - _D // 2), (0, _D - 1)))
    bp2p = jnp.broadcast_to(bp2, (_D,)).reshape(1, _D)

    a = _sc_agg(x, src2d, dst2d)
    x1 = _tc_mlp(x, a[0], a[1], W1_0, b1_0, W2_0, b2_0, ln_g_0, ln_b_0)
    a = _sc_agg(x1, src2d, dst2d)
    x2 = _tc_mlp(x1, a[0], a[1], W1_1, b1_1, W2_1, b2_1, ln_g_1, ln_b_1)
    a = _sc_agg(x2, src2d, dst2d)
    return _tc_final(x2, a[0], a[1], W1_2, b1_2, W2_2, b2_2, batch3d,
                     wp1p, bp1p, wp2p, bp2p)


# speedup vs baseline: 1.0232x; 1.0232x over previous
import functools

import jax
import jax.numpy as jnp
from jax import lax
from jax.experimental import pallas as pl
from jax.experimental.pallas import tpu as pltpu
from jax.experimental.pallas import tpu_sc as plsc

_N, _E, _D, _G = 10000, 320000, 128, 64
_NC, _NS = 2, 16
_NW = _NC * _NS
_K = 80
_NCHUNK = 128
_EPW = _NCHUNK * _K
_EPAD = _NW * _EPW
_P = 64
_NTRASH = 16
_NPAD = _N + _NTRASH
_STRIPE = 640
_R = 1000
_NBLK = _N // _R

_F32 = jnp.float32
_HI = lax.Precision.DEFAULT


def _sc_agg(x, src2d, dst2d):
    mesh = plsc.VectorSubcoreMesh(core_axis_name="c", subcore_axis_name="s")

    @functools.partial(
        pl.kernel,
        out_type=jax.ShapeDtypeStruct((_NC, _NPAD, _D), _F32),
        mesh=mesh,
        scratch_types=[
            pltpu.VMEM((_P, _K), jnp.int32),
            pltpu.VMEM((_P, _K), jnp.int32),
            pltpu.VMEM((_K, _D), _F32),
            pltpu.VMEM((_K, _D), _F32),
            pltpu.VMEM_SHARED((_NPAD, _D), _F32),
            pltpu.SemaphoreType.DMA,
            pltpu.SemaphoreType.DMA,
            pltpu.SemaphoreType.DMA,
            pltpu.SemaphoreType.DMA,
        ],
    )
    def agg_kernel(x_hbm, src_hbm, dst_hbm, out_hbm,
                   src_v, dst_v, rows_a, rows_b, acc_sh, g_a, g_b, s_a, s_b):
        cid = lax.axis_index("c")
        sid = lax.axis_index("s")
        wid = sid * _NC + cid
        rows = (rows_a, rows_b)
        gsem = (g_a, g_b)
        ssem = (s_a, s_b)

        @pl.loop(0, _K * (_D // 16))
        def _(i):
            r = i // (_D // 16)
            c = (i % (_D // 16)) * 16
            rows_a[r, pl.ds(c, 16)] = jnp.zeros((16,), _F32)

        base = sid * _STRIPE
        nchk = jnp.where(sid == _NS - 1,
                         (_NPAD - (_NS - 1) * _STRIPE) // 16, _STRIPE // 16)

        @pl.loop(0, nchk)
        def _(t):
            pltpu.sync_copy(rows_a.at[pl.ds(0, 16)],
                            acc_sh.at[pl.ds(base + t * 16, 16)])

        plsc.subcore_barrier()

        for p in range(_NCHUNK // _P):
            pltpu.sync_copy(src_hbm.at[wid, pl.ds(p * _P, _P)], src_v)
            pltpu.sync_copy(dst_hbm.at[wid, pl.ds(p * _P, _P)], dst_v)
            pltpu.async_copy(x_hbm.at[src_v.at[0]], rows_a, g_a)

            @pl.loop(0, _P // 2)
            def _(t):
                j = t * 2
                for b in range(2):
                    jb = j + b
                    pltpu.make_async_copy(x_hbm.at[src_v.at[jb]],
                                          rows[b], gsem[b]).wait()

                    @pl.when(jb + 1 < _P)
                    def _(b=b, jb=jb):
                        pltpu.async_copy(x_hbm.at[src_v.at[jb + 1]],
                                         rows[1 - b], gsem[1 - b])

                    pltpu.sync_copy(rows[b], acc_sh.at[dst_v.at[jb]], add=True)

        plsc.subcore_barrier()

        @pl.loop(0, nchk)
        def _(t):
            pltpu.sync_copy(acc_sh.at[pl.ds(base + t * 16, 16)],
                            out_hbm.at[cid, pl.ds(base + t * 16, 16)])

    return agg_kernel(x, src2d, dst2d)


def _mlp_body(x_ref, a0_ref, a1_ref, w1_ref, b1_ref, w2_ref, b2_ref,
              g_ref, be_ref, o_ref):
    h = x_ref[...] + a0_ref[...] + a1_ref[...]
    h = jnp.maximum(lax.dot(h, w1_ref[...], precision=_HI) + b1_ref[...], 0.0)
    h = lax.dot(h, w2_ref[...], precision=_HI) + b2_ref[...]
    h = jnp.maximum(h, 0.0)
    mu = jnp.mean(h, axis=-1, keepdims=True)
    var = jnp.mean((h - mu) ** 2, axis=-1, keepdims=True)
    o_ref[...] = (h - mu) / jnp.sqrt(var + 1e-5) * g_ref[...] + be_ref[...]


def _tc_mlp(x, a0, a1, w1, b1, w2, b2, g, be):
    row = pl.BlockSpec((_R, _D), lambda i: (i, 0))
    full = pl.BlockSpec((_D, _D), lambda i: (0, 0))
    vec = pl.BlockSpec((1, _D), lambda i: (0, 0))
    return pl.pallas_call(
        _mlp_body,
        grid=(_NBLK,),
        in_specs=[row, row, row, full, vec, full, vec, vec, vec],
        out_specs=row,
        out_shape=jax.ShapeDtypeStruct((_N, _D), _F32),
    )(x, a0, a1, w1, b1.reshape(1, _D), w2, b2.reshape(1, _D),
      g.reshape(1, _D), be.reshape(1, _D))


def _final_body(x_ref, a0_ref, a1_ref, w1_ref, b1_ref, w2_ref, b2_ref,
                batch_ref, wp1_ref, bp1_ref, wp2_ref, bp2_ref,
                o_ref, sums_s, cnt_s):
    i = pl.program_id(0)
    h = x_ref[...] + a0_ref[...] + a1_ref[...]
    h = jnp.maximum(lax.dot(h, w1_ref[...], precision=_HI) + b1_ref[...], 0.0)
    h = lax.dot(h, w2_ref[...], precision=_HI) + b2_ref[...]
    h = jnp.maximum(h, 0.0)

    bb = batch_ref[0]
    oh = (lax.broadcasted_iota(jnp.int32, (_G, _R), 0)
          == jnp.broadcast_to(bb, (_G, _R))).astype(_F32)
    ps = lax.dot(oh, h, precision=lax.Precision.HIGHEST)
    pc = jnp.broadcast_to(jnp.sum(oh, axis=1, keepdims=True), (_G, _D))

    @pl.when(i == 0)
    def _():
        sums_s[...] = ps
        cnt_s[...] = pc

    @pl.when(i > 0)
    def _():
        sums_s[...] += ps
        cnt_s[...] += pc

    @pl.when(i == _NBLK - 1)
    def _():
        pooled = sums_s[...] / jnp.maximum(cnt_s[...], 1.0)
        ph = jnp.maximum(lax.dot(pooled, wp1_ref[...], precision=_HI)
                         + bp1_ref[...], 0.0)
        o_ref[...] = lax.dot(ph, wp2_ref[...], precision=_HI) + bp2_ref[...]


def _tc_final(x, a0, a1, w1, b1, w2, b2, batch3d, wp1p, bp1p, wp2p, bp2p):
    row = pl.BlockSpec((_R, _D), lambda i: (i, 0))
    full = pl.BlockSpec((_D, _D), lambda i: (0, 0))
    vec = pl.BlockSpec((1, _D), lambda i: (0, 0))
    bspec = pl.BlockSpec((1, 1, _R), lambda i: (i, 0, 0))
    out = pl.pallas_call(
        _final_body,
        grid=(_NBLK,),
        in_specs=[row, row, row, full, vec, full, vec, bspec,
                  full, vec, full, vec],
        out_specs=pl.BlockSpec((_G, _D), lambda i: (0, 0)),
        out_shape=jax.ShapeDtypeStruct((_G, _D), _F32),
        scratch_shapes=[pltpu.VMEM((_G, _D), _F32), pltpu.VMEM((_G, _D), _F32)],
    )(x, a0, a1, w1, b1.reshape(1, _D), w2, b2.reshape(1, _D), batch3d,
      wp1p, bp1p, wp2p, bp2p)
    return out[:, :1]


def kernel(x, edge_index, batch,
           W1_0, b1_0, W2_0, b2_0,
           W1_1, b1_1, W2_1, b2_1,
           W1_2, b1_2, W2_2, b2_2,
           ln_g_0, ln_b_0, ln_g_1, ln_b_1,
           Wp1, bp1, Wp2, bp2):
    pad = _EPAD - _E
    src2d = jnp.concatenate(
        [edge_index[0], jnp.zeros((pad,), jnp.int32)]
    ).reshape(_NW, _NCHUNK, _K)
    dst2d = jnp.concatenate(
        [edge_index[1], _N + (jnp.arange(pad, dtype=jnp.int32) % _NTRASH)]
    ).reshape(_NW, _NCHUNK, _K)
    batch3d = batch.reshape(_NBLK, 1, _R)

    wp1p = jnp.pad(Wp1, ((0, 0), (0, _D - _D // 2)))
    bp1p = jnp.pad(bp1, (0, _D - _D // 2)).reshape(1, _D)
    wp2p = jnp.pad(Wp2, ((0, _D - _D // 2), (0, _D - 1)))
    bp2p = jnp.broadcast_to(bp2, (_D,)).reshape(1, _D)

    a = _sc_agg(x, src2d, dst2d)
    x1 = _tc_mlp(x, a[0], a[1], W1_0, b1_0, W2_0, b2_0, ln_g_0, ln_b_0)
    a = _sc_agg(x1, src2d, dst2d)
    x2 = _tc_mlp(x1, a[0], a[1], W1_1, b1_1, W2_1, b2_1, ln_g_1, ln_b_1)
    a = _sc_agg(x2, src2d, dst2d)
    return _tc_final(x2, a[0], a[1], W1_2, b1_2, W2_2, b2_2, batch3d,
                     wp1p, bp1p, wp2p, bp2p)


# speedup vs baseline: 2.1839x; 2.1344x over previous
import functools

import jax
import jax.numpy as jnp
from jax import lax
from jax.experimental import pallas as pl
from jax.experimental.pallas import tpu as pltpu
from jax.experimental.pallas import tpu_sc as plsc

_N, _E, _D, _G = 10000, 320000, 128, 64
_NC, _NS = 2, 16
_NW = _NC * _NS
_EPW = _E // _NW
_K = 80
_NCHUNK = _EPW // _K
_STRIPE = 640
_ZROWS = 80
_R = 1000
_NBLK = _N // _R

_F32 = jnp.float32
_HI = lax.Precision.DEFAULT


def _sc_agg(x, src2d, dst2d):
    mesh = plsc.VectorSubcoreMesh(core_axis_name="c", subcore_axis_name="s")

    @functools.partial(
        pl.kernel,
        out_type=jax.ShapeDtypeStruct((_NC, _N, _D), _F32),
        mesh=mesh,
        scratch_types=[
            pltpu.VMEM((_NCHUNK, _K), jnp.int32),
            pltpu.VMEM((_NCHUNK, _K), jnp.int32),
            pltpu.VMEM((_K, _D), _F32),
            pltpu.VMEM_SHARED((_N, _D), _F32),
            pltpu.SemaphoreType.DMA,
        ],
    )
    def agg_kernel(x_hbm, src_hbm, dst_hbm, out_hbm,
                   src_v, dst_v, rows_v, acc_sh, sem):
        cid = lax.axis_index("c")
        sid = lax.axis_index("s")
        wid = sid * _NC + cid

        @pl.loop(0, _ZROWS * (_D // 16))
        def _(i):
            r = i // (_D // 16)
            c = (i % (_D // 16)) * 16
            rows_v[r, pl.ds(c, 16)] = jnp.zeros((16,), _F32)

        base = sid * _STRIPE
        nchk = jnp.where(sid == _NS - 1, (_N - (_NS - 1) * _STRIPE) // _ZROWS,
                         _STRIPE // _ZROWS)

        @pl.loop(0, nchk)
        def _(t):
            pltpu.sync_copy(rows_v, acc_sh.at[pl.ds(base + t * _ZROWS, _ZROWS)])

        pltpu.sync_copy(src_hbm.at[wid], src_v)
        pltpu.sync_copy(dst_hbm.at[wid], dst_v)

        plsc.subcore_barrier()

        @pl.loop(0, _NCHUNK)
        def _(j):
            pltpu.async_copy(x_hbm.at[src_v.at[j]], rows_v, sem).wait()
            pltpu.sync_copy(rows_v, acc_sh.at[dst_v.at[j]], add=True)

        plsc.subcore_barrier()

        @pl.loop(0, nchk)
        def _(t):
            pltpu.sync_copy(acc_sh.at[pl.ds(base + t * _ZROWS, _ZROWS)],
                            out_hbm.at[cid, pl.ds(base + t * _ZROWS, _ZROWS)])

    return agg_kernel(x, src2d, dst2d)


def _mlp_body(x_ref, a0_ref, a1_ref, w1_ref, b1_ref, w2_ref, b2_ref,
              g_ref, be_ref, o_ref):
    h = x_ref[...] + a0_ref[...] + a1_ref[...]
    h = jnp.maximum(lax.dot(h, w1_ref[...], precision=_HI) + b1_ref[...], 0.0)
    h = lax.dot(h, w2_ref[...], precision=_HI) + b2_ref[...]
    h = jnp.maximum(h, 0.0)
    mu = jnp.mean(h, axis=-1, keepdims=True)
    var = jnp.mean((h - mu) ** 2, axis=-1, keepdims=True)
    o_ref[...] = (h - mu) / jnp.sqrt(var + 1e-5) * g_ref[...] + be_ref[...]


def _tc_mlp(x, a0, a1, w1, b1, w2, b2, g, be):
    row = pl.BlockSpec((_R, _D), lambda i: (i, 0))
    full = pl.BlockSpec((_D, _D), lambda i: (0, 0))
    vec = pl.BlockSpec((1, _D), lambda i: (0, 0))
    return pl.pallas_call(
        _mlp_body,
        grid=(_NBLK,),
        in_specs=[row, row, row, full, vec, full, vec, vec, vec],
        out_specs=row,
        out_shape=jax.ShapeDtypeStruct((_N, _D), _F32),
    )(x, a0, a1, w1, b1.reshape(1, _D), w2, b2.reshape(1, _D),
      g.reshape(1, _D), be.reshape(1, _D))


def _final_body(x_ref, a0_ref, a1_ref, w1_ref, b1_ref, w2_ref, b2_ref,
                batch_ref, wp1_ref, bp1_ref, wp2_ref, bp2_ref,
                o_ref, sums_s, cnt_s):
    i = pl.program_id(0)
    h = x_ref[...] + a0_ref[...] + a1_ref[...]
    h = jnp.maximum(lax.dot(h, w1_ref[...], precision=_HI) + b1_ref[...], 0.0)
    h = lax.dot(h, w2_ref[...], precision=_HI) + b2_ref[...]
    h = jnp.maximum(h, 0.0)

    bb = batch_ref[0]
    oh = (lax.broadcasted_iota(jnp.int32, (_G, _R), 0)
          == jnp.broadcast_to(bb, (_G, _R))).astype(_F32)
    ps = lax.dot(oh, h, precision=lax.Precision.HIGHEST)
    pc = jnp.broadcast_to(jnp.sum(oh, axis=1, keepdims=True), (_G, _D))

    @pl.when(i == 0)
    def _():
        sums_s[...] = ps
        cnt_s[...] = pc

    @pl.when(i > 0)
    def _():
        sums_s[...] += ps
        cnt_s[...] += pc

    @pl.when(i == _NBLK - 1)
    def _():
        pooled = sums_s[...] / jnp.maximum(cnt_s[...], 1.0)
        ph = jnp.maximum(lax.dot(pooled, wp1_ref[...], precision=_HI)
                         + bp1_ref[...], 0.0)
        o_ref[...] = lax.dot(ph, wp2_ref[...], precision=_HI) + bp2_ref[...]


def _tc_final(x, a0, a1, w1, b1, w2, b2, batch3d, wp1p, bp1p, wp2p, bp2p):
    row = pl.BlockSpec((_R, _D), lambda i: (i, 0))
    full = pl.BlockSpec((_D, _D), lambda i: (0, 0))
    vec = pl.BlockSpec((1, _D), lambda i: (0, 0))
    bspec = pl.BlockSpec((1, 1, _R), lambda i: (i, 0, 0))
    out = pl.pallas_call(
        _final_body,
        grid=(_NBLK,),
        in_specs=[row, row, row, full, vec, full, vec, bspec,
                  full, vec, full, vec],
        out_specs=pl.BlockSpec((_G, _D), lambda i: (0, 0)),
        out_shape=jax.ShapeDtypeStruct((_G, _D), _F32),
        scratch_shapes=[pltpu.VMEM((_G, _D), _F32), pltpu.VMEM((_G, _D), _F32)],
    )(x, a0, a1, w1, b1.reshape(1, _D), w2, b2.reshape(1, _D), batch3d,
      wp1p, bp1p, wp2p, bp2p)
    return out[:, :1]


def kernel(x, edge_index, batch,
           W1_0, b1_0, W2_0, b2_0,
           W1_1, b1_1, W2_1, b2_1,
           W1_2, b1_2, W2_2, b2_2,
           ln_g_0, ln_b_0, ln_g_1, ln_b_1,
           Wp1, bp1, Wp2, bp2):
    src2d = edge_index[0].reshape(_NW, _NCHUNK, _K)
    dst2d = edge_index[1].reshape(_NW, _NCHUNK, _K)
    batch3d = batch.reshape(_NBLK, 1, _R)

    wp1p = jnp.pad(Wp1, ((0, 0), (0, _D - _D // 2)))
    bp1p = jnp.pad(bp1, (0, _D - _D // 2)).reshape(1, _D)
    wp2p = jnp.pad(Wp2, ((0, _D - _D // 2), (0, _D - 1)))
    bp2p = jnp.broadcast_to(bp2, (_D,)).reshape(1, _D)

    a = _sc_agg(x, src2d, dst2d)
    x1 = _tc_mlp(x, a[0], a[1], W1_0, b1_0, W2_0, b2_0, ln_g_0, ln_b_0)
    a = _sc_agg(x1, src2d, dst2d)
    x2 = _tc_mlp(x1, a[0], a[1], W1_1, b1_1, W2_1, b2_1, ln_g_1, ln_b_1)
    a = _sc_agg(x2, src2d, dst2d)
    return _tc_final(x2, a[0], a[1], W1_2, b1_2, W2_2, b2_2, batch3d,
                     wp1p, bp1p, wp2p, bp2p)
